# baseline (speedup 1.0000x reference)
import functools

import jax
import jax.numpy as jnp
from jax import lax
from jax.experimental import pallas as pl
from jax.experimental.pallas import tpu as pltpu
from jax.experimental.pallas import tpu_sc as plsc

BATCH = 16384
D = 64
NC = 2
NS = 16
L = 16
NW = NC * NS
BPW = BATCH // NW
NCHUNK = 4
CHUNK = BPW // NCHUNK
GROUPS = BPW // L


def _mf_body(uid_hbm, iid_hbm, ut_hbm, it_hbm, wb_hbm, out_hbm,
             uidx_v, iidx_v, urows_v, irows_v, out_v, wb_v, sem_u, sem_i):
    wid = lax.axis_index("s") * NC + lax.axis_index("c")
    base = wid * BPW

    for j in range(NCHUNK):
        pltpu.sync_copy(uid_hbm.at[pl.ds(base + j * CHUNK, CHUNK)], uidx_v.at[j])
        pltpu.sync_copy(iid_hbm.at[pl.ds(base + j * CHUNK, CHUNK)], iidx_v.at[j])
    pltpu.sync_copy(wb_hbm, wb_v)

    cps = []
    for j in range(NCHUNK):
        cps.append(pltpu.async_copy(
            ut_hbm.at[uidx_v.at[j]], urows_v.at[pl.ds(j * CHUNK, CHUNK)], sem_u))
        cps.append(pltpu.async_copy(
            it_hbm.at[iidx_v.at[j]], irows_v.at[pl.ds(j * CHUNK, CHUNK)], sem_i))
    for c in cps:
        c.wait()

    w_vec = wb_v[0]
    b_vec = wb_v[1]

    def group(g, carry):
        row0 = g * L
        rows = row0 + lax.iota(jnp.int32, L)
        acc = jnp.zeros((L,), jnp.float32)
        for c in range(D):
            cols = jnp.full((L,), c, jnp.int32)
            u = plsc.load_gather(urows_v, [rows, cols])
            v = plsc.load_gather(irows_v, [rows, cols])
            acc = acc + u * v
        out_v[pl.ds(row0, L)] = acc * w_vec + b_vec
        return carry

    lax.fori_loop(0, GROUPS, group, 0)
    pltpu.sync_copy(out_v, out_hbm.at[pl.ds(base, BPW)])


@functools.partial(
    pl.kernel,
    mesh=plsc.VectorSubcoreMesh(core_axis_name="c", subcore_axis_name="s"),
    compiler_params=pltpu.CompilerParams(use_tc_tiling_on_sc=False,
                                         needs_layout_passes=False),
    out_type=jax.ShapeDtypeStruct((BATCH,), jnp.float32),
    scratch_types=[
        pltpu.VMEM((NCHUNK, CHUNK), jnp.int32),
        pltpu.VMEM((NCHUNK, CHUNK), jnp.int32),
        pltpu.VMEM((BPW, D), jnp.float32),
        pltpu.VMEM((BPW, D), jnp.float32),
        pltpu.VMEM((BPW,), jnp.float32),
        pltpu.VMEM((2, L), jnp.float32),
        pltpu.SemaphoreType.DMA,
        pltpu.SemaphoreType.DMA,
    ],
)
def _mf_sc(*args):
    _mf_body(*args)


def kernel(user_ids, item_ids, user_table, item_table, W, b):
    uid = user_ids.astype(jnp.int32)
    iid = item_ids.astype(jnp.int32)
    wb = jnp.stack([jnp.full((L,), W[0, 0], jnp.float32),
                    jnp.full((L,), b[0], jnp.float32)])
    out = _mf_sc(uid, iid, user_table, item_table, wb)
    return out.reshape(BATCH, 1)


# speedup vs baseline: 1.5384x; 1.5384x over previous
import functools

import jax
import jax.numpy as jnp
from jax import lax
from jax.experimental import pallas as pl
from jax.experimental.pallas import tpu as pltpu
from jax.experimental.pallas import tpu_sc as plsc

BATCH = 16384
D = 64
NC = 2
NS = 16
L = 16
NW = NC * NS
BPW = BATCH // NW
NCHUNK = 4
CHUNK = BPW // NCHUNK
GPC = CHUNK // L
GROUPS = BPW // L
PASSES = 2
SPP = BPW // PASSES


def _mf_body(uid_hbm, iid_hbm, ut_hbm, it_hbm, wb_hbm, out_hbm, dummy_hbm,
             uidx_v, iidx_v, urows_v, irows_v, out_v, wb_v, sem_u, sem_i):
    wid = lax.axis_index("s") * NC + lax.axis_index("c")
    base = wid * BPW

    for j in range(NCHUNK):
        pltpu.sync_copy(uid_hbm.at[pl.ds(base + j * CHUNK, CHUNK)], uidx_v.at[j])
        pltpu.sync_copy(iid_hbm.at[pl.ds(base + j * CHUNK, CHUNK)], iidx_v.at[j])
    pltpu.sync_copy(wb_hbm, wb_v)

    w_vec = wb_v[pl.ds(0, L)]
    b_vec = wb_v[pl.ds(L, L)]

    for p in range(PASSES):
        for j in range(NCHUNK // PASSES):
            jj = p * (NCHUNK // PASSES) + j
            def stage(g, carry, j=j, jj=jj):
                uids = uidx_v[jj, pl.ds(g * L, L)]
                iids = iidx_v[jj, pl.ds(g * L, L)]
                brow0 = j * CHUNK + g * L
                for k in range(L):
                    brow = brow0 + k
                    pltpu.async_copy(
                        ut_hbm.at[pl.ds(uids[k], 1), :],
                        urows_v.at[pl.ds(brow, 1), :], sem_u)
                    pltpu.async_copy(
                        it_hbm.at[pl.ds(iids[k], 1), :],
                        irows_v.at[pl.ds(brow, 1), :], sem_i)
                return carry
            lax.fori_loop(0, GPC, stage, 0)

        pltpu.make_async_copy(dummy_hbm, urows_v, sem_u).wait()
        pltpu.make_async_copy(dummy_hbm, irows_v, sem_i).wait()

        def group(g, carry, p=p):
            rows = g * L + lax.iota(jnp.int32, L)
            acc = jnp.zeros((L,), jnp.float32)
            for c in range(D):
                cols = jnp.full((L,), c, jnp.int32)
                u = plsc.load_gather(urows_v, [rows, cols])
                v = plsc.load_gather(irows_v, [rows, cols])
                acc = acc + u * v
            out_v[pl.ds(p * SPP + g * L, L)] = acc * w_vec + b_vec
            return carry

        lax.fori_loop(0, SPP // L, group, 0)

    pltpu.sync_copy(out_v, out_hbm.at[pl.ds(base, BPW)])


@functools.partial(
    pl.kernel,
    mesh=plsc.VectorSubcoreMesh(core_axis_name="c", subcore_axis_name="s"),
    compiler_params=pltpu.CompilerParams(needs_layout_passes=False),
    out_type=(jax.ShapeDtypeStruct((BATCH,), jnp.float32),
              jax.ShapeDtypeStruct((SPP, D), jnp.float32)),
    scratch_types=[
        pltpu.VMEM((NCHUNK, CHUNK), jnp.int32),
        pltpu.VMEM((NCHUNK, CHUNK), jnp.int32),
        pltpu.VMEM((SPP, D), jnp.float32),
        pltpu.VMEM((SPP, D), jnp.float32),
        pltpu.VMEM((BPW,), jnp.float32),
        pltpu.VMEM((2 * L,), jnp.float32),
        pltpu.SemaphoreType.DMA,
        pltpu.SemaphoreType.DMA,
    ],
)
def _mf_sc(*args):
    _mf_body(*args)


def kernel(user_ids, item_ids, user_table, item_table, W, b):
    uid = user_ids.astype(jnp.int32)
    iid = item_ids.astype(jnp.int32)
    wb = jnp.concatenate([jnp.full((L,), W[0, 0], jnp.float32),
                          jnp.full((L,), b[0], jnp.float32)])
    out, _ = _mf_sc(uid, iid, user_table, item_table, wb)
    return out.reshape(BATCH, 1)
